# baseline (speedup 1.0000x reference)
import functools

import jax
import jax.numpy as jnp
from jax import lax
from jax.experimental import pallas as pl
from jax.experimental.pallas import tpu as pltpu
from jax.experimental.pallas import tpu_sc as plsc

_TOPK = 8
_NE = 64
_BLOCK = 512
_NW = 32
_TPW = 16384 // _NW


def _logits_kernel(x_ref, w_ref, b_ref, out_ref):
    x = x_ref[...]
    w = w_ref[...]
    out_ref[...] = (
        jax.lax.dot_general(
            x, w, (((1,), (1,)), ((), ())), preferred_element_type=jnp.float32
        )
        + b_ref[...]
    )


def _sc_topk(logits_hbm, wts_hbm, idx_hbm, lg_v, w_v, i_v):
    wid = lax.axis_index("s") * 2 + lax.axis_index("c")
    base = wid * _TPW
    pltpu.sync_copy(logits_hbm.at[pl.ds(base * _NE, _TPW * _NE)], lg_v)

    lanes = lax.iota(jnp.int32, 16)
    neg_inf = jnp.full((16,), -jnp.inf, dtype=jnp.float32)

    def group_body(g, _):
        toks = g * 16 + lanes

        def chunk_body(c, tops):
            tops = list(tops)
            for u in range(8):
                e = c * 8 + u
                v = plsc.load_gather(lg_v, [toks * _NE + e])
                bits = lax.bitcast_convert_type(v, jnp.int32)
                code = jnp.where(
                    bits >= 0, jnp.int32(_NE - 1) - e, jnp.int32(0) + e
                )
                cand = lax.bitcast_convert_type(
                    (bits & jnp.int32(-64)) | code, jnp.float32
                )
                for i in range(_TOPK):
                    hi = jnp.maximum(tops[i], cand)
                    cand = jnp.minimum(tops[i], cand)
                    tops[i] = hi
            return tuple(tops)

        tops = lax.fori_loop(
            0, _NE // 8, chunk_body, (neg_inf,) * _TOPK, unroll=False
        )

        vals = []
        idxs = []
        for i in range(_TOPK):
            mb = lax.bitcast_convert_type(tops[i], jnp.int32)
            low = mb & jnp.int32(_NE - 1)
            idxs.append(jnp.where(mb >= 0, jnp.int32(_NE - 1) - low, low))
            vals.append(
                lax.bitcast_convert_type(mb & jnp.int32(-64), jnp.float32)
            )
        es = [jnp.exp(v - vals[0]) for v in vals]
        tot = es[0]
        for i in range(1, _TOPK):
            tot = tot + es[i]
        for i in range(_TOPK):
            pos = toks * _TOPK + i
            plsc.store_scatter(w_v, [pos], es[i] / tot)
            plsc.store_scatter(i_v, [pos], idxs[i])
        return 0

    lax.fori_loop(0, _TPW // 16, group_body, 0, unroll=False)

    pltpu.sync_copy(w_v, wts_hbm.at[pl.ds(base * _TOPK, _TPW * _TOPK)])
    pltpu.sync_copy(i_v, idx_hbm.at[pl.ds(base * _TOPK, _TPW * _TOPK)])


def kernel(x, W, b):
    n, d = x.shape
    logits = pl.pallas_call(
        _logits_kernel,
        grid=(n // _BLOCK,),
        in_specs=[
            pl.BlockSpec((_BLOCK, d), lambda i: (i, 0)),
            pl.BlockSpec((_NE, d), lambda i: (0, 0)),
            pl.BlockSpec((1, _NE), lambda i: (0, 0)),
        ],
        out_specs=pl.BlockSpec((_BLOCK, _NE), lambda i: (i, 0)),
        out_shape=jax.ShapeDtypeStruct((n, _NE), jnp.float32),
        compiler_params=pltpu.CompilerParams(
            dimension_semantics=("parallel",),
        ),
    )(x, W, b.reshape(1, _NE))

    mesh = plsc.VectorSubcoreMesh(
        core_axis_name="c", subcore_axis_name="s", num_cores=2, num_subcores=16
    )
    wts, idx = pl.kernel(
        _sc_topk,
        out_type=[
            jax.ShapeDtypeStruct((n * _TOPK,), jnp.float32),
            jax.ShapeDtypeStruct((n * _TOPK,), jnp.int32),
        ],
        mesh=mesh,
        scratch_types=[
            pltpu.VMEM((_TPW * _NE,), jnp.float32),
            pltpu.VMEM((_TPW * _TOPK,), jnp.float32),
            pltpu.VMEM((_TPW * _TOPK,), jnp.int32),
        ],
        compiler_params=pltpu.CompilerParams(needs_layout_passes=False),
    )(logits.reshape(n * _NE))
    return wts.reshape(n, _TOPK), idx.reshape(n, _TOPK)


# speedup vs baseline: 1.3468x; 1.3468x over previous
import jax
import jax.numpy as jnp
from jax.experimental import pallas as pl
from jax.experimental.pallas import tpu as pltpu

_TOPK = 8
_NE = 64
_BLOCK = 512
_CHUNK = 64


def _topk_tail(logits, wts_ref, idx_ref):
    for c in range(_BLOCK // _CHUNK):
        lg = logits[c * _CHUNK:(c + 1) * _CHUNK, :]
        bits = jax.lax.bitcast_convert_type(lg, jnp.int32)
        cols = jax.lax.broadcasted_iota(jnp.int32, lg.shape, 1)
        code = jnp.where(bits >= 0, jnp.int32(_NE - 1) - cols, cols)
        key = jax.lax.bitcast_convert_type(
            (bits & jnp.int32(-64)) | code, jnp.float32
        )

        neg_inf = jnp.float32(-jnp.inf)
        tops = []
        for _ in range(_TOPK):
            m = jnp.max(key, axis=-1, keepdims=True)
            tops.append(m)
            key = jnp.where(key == m, neg_inf, key)

        mf = jnp.concatenate(tops, axis=-1)
        mb = jax.lax.bitcast_convert_type(mf, jnp.int32)
        low = mb & jnp.int32(_NE - 1)
        tidx = jnp.where(mb >= 0, jnp.int32(_NE - 1) - low, low)
        top = jax.lax.bitcast_convert_type(mb & jnp.int32(-64), jnp.float32)

        e = jnp.exp(top - top[:, 0:1])
        wts_ref[c * _CHUNK:(c + 1) * _CHUNK, :] = e / jnp.sum(
            e, axis=-1, keepdims=True
        )
        idx_ref[c * _CHUNK:(c + 1) * _CHUNK, :] = tidx


def _gate_kernel(x_ref, w_ref, b_ref, wts_ref, idx_ref, lg_ref):
    i = pl.program_id(0)
    nb = pl.num_programs(0)

    @pl.when(i > 0)
    def _tail():
        par = jax.lax.rem(i + 1, 2)
        _topk_tail(lg_ref[par], wts_ref, idx_ref)

    @pl.when(i < nb - 1)
    def _mm():
        par = jax.lax.rem(i, 2)
        x = x_ref[...]
        w = w_ref[...]
        lg_ref[par] = (
            jax.lax.dot_general(
                x, w, (((1,), (1,)), ((), ())),
                preferred_element_type=jnp.float32,
            )
            + b_ref[...]
        )


def kernel(x, W, b):
    n, d = x.shape
    nb = n // _BLOCK
    wts, idx = pl.pallas_call(
        _gate_kernel,
        grid=(nb + 1,),
        in_specs=[
            pl.BlockSpec((_BLOCK, d), lambda i: (jnp.minimum(i, nb - 1), 0)),
            pl.BlockSpec((_NE, d), lambda i: (0, 0)),
            pl.BlockSpec((1, _NE), lambda i: (0, 0)),
        ],
        out_specs=[
            pl.BlockSpec(
                (_BLOCK, _TOPK), lambda i: (jnp.maximum(i - 1, 0), 0)
            ),
            pl.BlockSpec(
                (_BLOCK, _TOPK), lambda i: (jnp.maximum(i - 1, 0), 0)
            ),
        ],
        out_shape=[
            jax.ShapeDtypeStruct((n, _TOPK), jnp.float32),
            jax.ShapeDtypeStruct((n, _TOPK), jnp.int32),
        ],
        scratch_shapes=[pltpu.VMEM((2, _BLOCK, _NE), jnp.float32)],
        compiler_params=pltpu.CompilerParams(
            dimension_semantics=("arbitrary",),
        ),
    )(x, W, b.reshape(1, _NE))
    return wts, idx
